# baseline (speedup 1.0000x reference)
import functools

import jax
import jax.numpy as jnp
from jax import lax
from jax.experimental import pallas as pl
from jax.experimental.pallas import tpu as pltpu
from jax.experimental.pallas import tpu_sc as plsc

B, L = 4096, 20
EMB = 64
WIDE = 128
C = 235
N = B * L
GRP = 8

NUM_CORES = 2
NUM_SUBCORES = 16
NW = NUM_CORES * NUM_SUBCORES
PER_W = N // NW
IDXCOLS = 128
CHUNK = 32
NCH = PER_W // CHUNK

BBLKA = 2048
NBA = B // BBLKA
BBLK = 512
NB = B // BBLK


def _gather_body(table_hbm, idx_hbm, out_hbm, idxbuf, gidx, groups, comp, gsem):
    wid = lax.axis_index("s") * NUM_CORES + lax.axis_index("c")
    row0 = wid * PER_W
    pltpu.sync_copy(idx_hbm.at[wid], idxbuf)

    def chunk_vecs(c):
        row = lax.div(c, IDXCOLS // CHUNK)
        off = lax.rem(c, IDXCOLS // CHUNK) * CHUNK
        return (idxbuf[row, pl.ds(off, 16)], idxbuf[row, pl.ds(off + 16, 16)])

    def fire(c, u):
        va, vb = chunk_vecs(c)
        gidx[u][pl.ds(0, 16)] = lax.shift_right_logical(va, 4)
        gidx[u][pl.ds(16, 16)] = lax.shift_right_logical(vb, 4)
        pltpu.async_copy(table_hbm.at[gidx[u]], groups[u], gsem[u])

    def drain(c, u):
        pltpu.make_async_copy(
            table_hbm.at[gidx[u]], groups[u], gsem[u]).wait()
        va, vb = chunk_vecs(c)
        ra = lax.bitwise_and(lax.shift_right_logical(va, 1), 7)
        rb = lax.bitwise_and(lax.shift_right_logical(vb, 1), 7)
        ha = lax.bitwise_and(va, 1) * EMB
        hb = lax.bitwise_and(vb, 1) * EMB
        for k in range(16):
            for q in range(EMB // 16):
                comp[u][k, pl.ds(q * 16, 16)] = (
                    groups[u][k, ra[k], pl.ds(ha[k] + q * 16, 16)])
                comp[u][16 + k, pl.ds(q * 16, 16)] = (
                    groups[u][16 + k, rb[k], pl.ds(hb[k] + q * 16, 16)])
        pltpu.sync_copy(comp[u], out_hbm.at[pl.ds(row0 + c * CHUNK, CHUNK)])

    fire(0, 0)

    def body(h, carry):
        c0 = 2 * h
        fire(c0 + 1, 1)
        drain(c0, 0)

        @pl.when(c0 + 2 < NCH)
        def _next():
            fire(c0 + 2, 0)

        drain(c0 + 1, 1)
        return carry

    lax.fori_loop(0, NCH // 2, body, 0)


@functools.cache
def _make_sc_gather():
    return pl.kernel(
        _gather_body,
        out_type=jax.ShapeDtypeStruct((N, WIDE), jnp.float32),
        mesh=plsc.VectorSubcoreMesh(core_axis_name="c", subcore_axis_name="s"),
        scratch_types=[
            pltpu.VMEM((PER_W // IDXCOLS, IDXCOLS), jnp.int32),
            [pltpu.VMEM((CHUNK,), jnp.int32)] * 2,
            [pltpu.VMEM((CHUNK, GRP, WIDE), jnp.float32)] * 2,
            [pltpu.VMEM((CHUNK, WIDE), jnp.float32)] * 2,
            [pltpu.SemaphoreType.DMA] * 2,
        ],
    )


def _lse_kernel(emb_ref, wt_ref, out_ref, m_ref, s_ref):
    l = pl.program_id(0)
    i = pl.program_id(1)

    @pl.when(i == 0)
    def _init():
        m_ref[...] = jnp.full((1, C), -jnp.inf, dtype=jnp.float32)
        s_ref[...] = jnp.zeros((1, C), dtype=jnp.float32)

    x = lax.dot_general(emb_ref[...], wt_ref[...], (((1,), (0,)), ((), ())),
                        preferred_element_type=jnp.float32)
    bm = jnp.max(x, axis=0, keepdims=True)
    bs = jnp.sum(jnp.exp(x - bm), axis=0, keepdims=True)
    m_old = m_ref[...]
    s_old = s_ref[...]
    m_new = jnp.maximum(m_old, bm)
    s_ref[...] = s_old * jnp.exp(m_old - m_new) + bs * jnp.exp(bm - m_new)
    m_ref[...] = m_new

    @pl.when(i == NBA - 1)
    def _fin():
        out_ref[pl.ds(l, 1), :] = m_ref[...] + jnp.log(s_ref[...])


def _out_kernel(emb_hbm, wt_ref, lse_ref, out_ref, ebuf, esem):
    i = pl.program_id(0)
    for l in range(L):
        pltpu.make_async_copy(
            emb_hbm.at[pl.ds(l * B + i * BBLK, BBLK)], ebuf.at[l],
            esem.at[l]).start()
    wt = wt_ref[...]
    for l in range(L):
        pltpu.make_async_copy(
            emb_hbm.at[pl.ds(l * B + i * BBLK, BBLK)], ebuf.at[l],
            esem.at[l]).wait()
        x = lax.dot_general(ebuf[l], wt, (((1,), (0,)), ((), ())),
                            preferred_element_type=jnp.float32)
        lse_row = lse_ref[pl.ds(l, 1), :]
        out_ref[:, pl.ds(l, 1), :] = (x - lse_row)[:, None, :]


def kernel(input, emb_weight, fc_weight, fc_bias):
    idx_l = input.astype(jnp.int32).T.reshape(-1)
    idx3 = idx_l.reshape(NW, PER_W // IDXCOLS, IDXCOLS)
    table3 = emb_weight.reshape(-1, GRP, WIDE)

    emb_l = _make_sc_gather()(table3, idx3)
    wt128 = jnp.concatenate(
        [fc_weight.T, jnp.zeros((WIDE - EMB, C), jnp.float32)], axis=0)

    lse = pl.pallas_call(
        _lse_kernel,
        grid=(L, NBA),
        in_specs=[
            pl.BlockSpec((BBLKA, WIDE), lambda l, i: (l * NBA + i, 0)),
            pl.BlockSpec((WIDE, C), lambda l, i: (0, 0)),
        ],
        out_specs=pl.BlockSpec((L, C), lambda l, i: (0, 0)),
        out_shape=jax.ShapeDtypeStruct((L, C), jnp.float32),
        scratch_shapes=[
            pltpu.VMEM((1, C), jnp.float32),
            pltpu.VMEM((1, C), jnp.float32),
        ],
    )(emb_l, wt128)

    out = pl.pallas_call(
        _out_kernel,
        grid=(NB,),
        in_specs=[
            pl.BlockSpec(memory_space=pl.ANY),
            pl.BlockSpec((WIDE, C), lambda i: (0, 0)),
            pl.BlockSpec((L, C), lambda i: (0, 0)),
        ],
        out_specs=pl.BlockSpec((BBLK, L, C), lambda i: (i, 0, 0)),
        out_shape=jax.ShapeDtypeStruct((B, L, C), jnp.float32),
        scratch_shapes=[
            pltpu.VMEM((L, BBLK, WIDE), jnp.float32),
            pltpu.SemaphoreType.DMA((L,)),
        ],
    )(emb_l, wt128, lse)
    return out


# speedup vs baseline: 1.6775x; 1.6775x over previous
import functools

import jax
import jax.numpy as jnp
from jax import lax
from jax.experimental import pallas as pl
from jax.experimental.pallas import tpu as pltpu
from jax.experimental.pallas import tpu_sc as plsc

B, L = 4096, 20
EMB = 64
C = 235
N = B * L
GRP = 8

NUM_CORES = 2
NUM_SUBCORES = 16
NW = NUM_CORES * NUM_SUBCORES
PER_W = N // NW
IDXCOLS = 128
CHUNK = 32
NCH = PER_W // CHUNK

BBLKA = 2048
NBA = B // BBLKA
BBLK = 512
NB = B // BBLK


def _gather_body(table_hbm, idx_hbm, out_hbm,
                 idxbuf, groups, comp, gsem, ssem):
    wid = lax.axis_index("s") * NUM_CORES + lax.axis_index("c")
    row0 = wid * PER_W
    pltpu.sync_copy(idx_hbm.at[wid], idxbuf)

    def chunk_vecs(c):
        row = lax.div(c, IDXCOLS // CHUNK)
        off = lax.rem(c, IDXCOLS // CHUNK) * CHUNK
        return (idxbuf[row, pl.ds(off, 16)], idxbuf[row, pl.ds(off + 16, 16)])

    def fire(c, u):
        va, vb = chunk_vecs(c)
        ga = lax.shift_right_logical(va, 3)
        gb = lax.shift_right_logical(vb, 3)
        for k in range(16):
            pltpu.async_copy(table_hbm.at[ga[k]], groups[u].at[k], gsem[u])
            pltpu.async_copy(table_hbm.at[gb[k]], groups[u].at[16 + k], gsem[u])

    def drain(c, u):
        for k in range(CHUNK):
            pltpu.make_async_copy(
                table_hbm.at[0], groups[u].at[k], gsem[u]).wait()
        va, vb = chunk_vecs(c)
        ra = lax.bitwise_and(va, 7)
        rb = lax.bitwise_and(vb, 7)
        for k in range(16):
            for q in range(EMB // 16):
                comp[u][k, pl.ds(q * 16, 16)] = (
                    groups[u][k, ra[k], pl.ds(q * 16, 16)])
                comp[u][16 + k, pl.ds(q * 16, 16)] = (
                    groups[u][16 + k, rb[k], pl.ds(q * 16, 16)])
        pltpu.sync_copy(comp[u], out_hbm.at[pl.ds(row0 + c * CHUNK, CHUNK)])

    fire(0, 0)

    def body(h, carry):
        c0 = 2 * h
        fire(c0 + 1, 1)
        drain(c0, 0)

        @pl.when(c0 + 2 < NCH)
        def _next():
            fire(c0 + 2, 0)

        drain(c0 + 1, 1)
        return carry

    lax.fori_loop(0, NCH // 2, body, 0)


@functools.cache
def _make_sc_gather():
    return pl.kernel(
        _gather_body,
        out_type=jax.ShapeDtypeStruct((N, EMB), jnp.float32),
        mesh=plsc.VectorSubcoreMesh(core_axis_name="c", subcore_axis_name="s"),
        scratch_types=[
            pltpu.VMEM((PER_W // IDXCOLS, IDXCOLS), jnp.int32),
            [pltpu.VMEM((CHUNK, GRP, EMB), jnp.float32)] * 2,
            [pltpu.VMEM((CHUNK, EMB), jnp.float32)] * 2,
            [pltpu.SemaphoreType.DMA] * 2,
            [pltpu.SemaphoreType.DMA] * 2,
        ],
    )


def _lse_kernel(emb_ref, wt_ref, out_ref, m_ref, s_ref):
    l = pl.program_id(0)
    i = pl.program_id(1)

    @pl.when(i == 0)
    def _init():
        m_ref[...] = jnp.full((1, C), -jnp.inf, dtype=jnp.float32)
        s_ref[...] = jnp.zeros((1, C), dtype=jnp.float32)

    x = lax.dot_general(emb_ref[...], wt_ref[...], (((1,), (0,)), ((), ())),
                        preferred_element_type=jnp.float32)
    bm = jnp.max(x, axis=0, keepdims=True)
    bs = jnp.sum(jnp.exp(x - bm), axis=0, keepdims=True)
    m_old = m_ref[...]
    s_old = s_ref[...]
    m_new = jnp.maximum(m_old, bm)
    s_ref[...] = s_old * jnp.exp(m_old - m_new) + bs * jnp.exp(bm - m_new)
    m_ref[...] = m_new

    @pl.when(i == NBA - 1)
    def _fin():
        out_ref[pl.ds(l, 1), :] = m_ref[...] + jnp.log(s_ref[...])


def _out_kernel(emb_hbm, wt_ref, lse_ref, out_ref, ebuf, esem):
    i = pl.program_id(0)
    for l in range(L):
        pltpu.make_async_copy(
            emb_hbm.at[pl.ds(l * B + i * BBLK, BBLK)], ebuf.at[l],
            esem.at[l]).start()
    wt = wt_ref[...]
    for l in range(L):
        pltpu.make_async_copy(
            emb_hbm.at[pl.ds(l * B + i * BBLK, BBLK)], ebuf.at[l],
            esem.at[l]).wait()
        x = lax.dot_general(ebuf[l], wt, (((1,), (0,)), ((), ())),
                            preferred_element_type=jnp.float32)
        lse_row = lse_ref[pl.ds(l, 1), :]
        out_ref[:, pl.ds(l, 1), :] = (x - lse_row)[:, None, :]


def kernel(input, emb_weight, fc_weight, fc_bias):
    idx_l = input.astype(jnp.int32).T.reshape(-1)
    idx3 = idx_l.reshape(NW, PER_W // IDXCOLS, IDXCOLS)
    table3 = emb_weight.reshape(-1, GRP, EMB)

    emb_l = _make_sc_gather()(table3, idx3)
    wt = fc_weight.T

    lse = pl.pallas_call(
        _lse_kernel,
        grid=(L, NBA),
        in_specs=[
            pl.BlockSpec((BBLKA, EMB), lambda l, i: (l * NBA + i, 0)),
            pl.BlockSpec((EMB, C), lambda l, i: (0, 0)),
        ],
        out_specs=pl.BlockSpec((L, C), lambda l, i: (0, 0)),
        out_shape=jax.ShapeDtypeStruct((L, C), jnp.float32),
        scratch_shapes=[
            pltpu.VMEM((1, C), jnp.float32),
            pltpu.VMEM((1, C), jnp.float32),
        ],
    )(emb_l, wt)

    out = pl.pallas_call(
        _out_kernel,
        grid=(NB,),
        in_specs=[
            pl.BlockSpec(memory_space=pl.ANY),
            pl.BlockSpec((EMB, C), lambda i: (0, 0)),
            pl.BlockSpec((L, C), lambda i: (0, 0)),
        ],
        out_specs=pl.BlockSpec((BBLK, L, C), lambda i: (i, 0, 0)),
        out_shape=jax.ShapeDtypeStruct((B, L, C), jnp.float32),
        scratch_shapes=[
            pltpu.VMEM((L, BBLK, EMB), jnp.float32),
            pltpu.SemaphoreType.DMA((L,)),
        ],
    )(emb_l, wt, lse)
    return out


# speedup vs baseline: 1.9963x; 1.1901x over previous
import functools

import jax
import jax.numpy as jnp
from jax import lax
from jax.experimental import pallas as pl
from jax.experimental.pallas import tpu as pltpu
from jax.experimental.pallas import tpu_sc as plsc

B, L = 4096, 20
EMB = 64
C = 235
N = B * L
GRP = 8

NUM_CORES = 2
NUM_SUBCORES = 16
NW = NUM_CORES * NUM_SUBCORES
PER_W = N // NW
IDXCOLS = 128
CHUNK = 32
NCH = PER_W // CHUNK

BBLKA = 2048
NBA = B // BBLKA
BBLK = 512
NB = B // BBLK


def _gather_body(table_hbm, idx_hbm, out_hbm,
                 idxbuf, groups, comp, gsem, ssem):
    wid = lax.axis_index("s") * NUM_CORES + lax.axis_index("c")
    row0 = wid * PER_W
    pltpu.sync_copy(idx_hbm.at[wid], idxbuf)

    def chunk_vecs(c):
        row = lax.div(c, IDXCOLS // CHUNK)
        off = lax.rem(c, IDXCOLS // CHUNK) * CHUNK
        return (idxbuf[row, pl.ds(off, 16)], idxbuf[row, pl.ds(off + 16, 16)])

    def fire(c, u):
        va, vb = chunk_vecs(c)
        ga = lax.shift_right_logical(va, 3)
        gb = lax.shift_right_logical(vb, 3)
        for k in range(16):
            pltpu.async_copy(table_hbm.at[ga[k]], groups[u].at[k], gsem[u])
            pltpu.async_copy(table_hbm.at[gb[k]], groups[u].at[16 + k], gsem[u])

    def drain(c, u):
        for k in range(CHUNK):
            pltpu.make_async_copy(
                table_hbm.at[0], groups[u].at[k], gsem[u]).wait()
        va, vb = chunk_vecs(c)
        ra = lax.bitwise_and(va, 7)
        rb = lax.bitwise_and(vb, 7)
        for k in range(16):
            for q in range(EMB // 16):
                comp[u][k, pl.ds(q * 16, 16)] = (
                    groups[u][k, ra[k], pl.ds(q * 16, 16)])
                comp[u][16 + k, pl.ds(q * 16, 16)] = (
                    groups[u][16 + k, rb[k], pl.ds(q * 16, 16)])
        pltpu.sync_copy(comp[u], out_hbm.at[pl.ds(row0 + c * CHUNK, CHUNK)])

    fire(0, 0)

    def body(h, carry):
        c0 = 2 * h
        fire(c0 + 1, 1)
        drain(c0, 0)

        @pl.when(c0 + 2 < NCH)
        def _next():
            fire(c0 + 2, 0)

        drain(c0 + 1, 1)
        return carry

    lax.fori_loop(0, NCH // 2, body, 0)


@functools.cache
def _make_sc_gather():
    return pl.kernel(
        _gather_body,
        out_type=jax.ShapeDtypeStruct((N, EMB), jnp.float32),
        mesh=plsc.VectorSubcoreMesh(core_axis_name="c", subcore_axis_name="s"),
        scratch_types=[
            pltpu.VMEM((PER_W // IDXCOLS, IDXCOLS), jnp.int32),
            [pltpu.VMEM((CHUNK, GRP, EMB), jnp.float32)] * 2,
            [pltpu.VMEM((CHUNK, EMB), jnp.float32)] * 2,
            [pltpu.SemaphoreType.DMA] * 2,
            [pltpu.SemaphoreType.DMA] * 2,
        ],
    )


def _lse_kernel(emb_ref, wt_ref, out_ref, m_ref, s_ref):
    l = pl.program_id(0)
    i = pl.program_id(1)

    @pl.when(i == 0)
    def _init():
        m_ref[...] = jnp.full((1, C), -jnp.inf, dtype=jnp.float32)
        s_ref[...] = jnp.zeros((1, C), dtype=jnp.float32)

    x = lax.dot_general(emb_ref[...], wt_ref[...], (((1,), (0,)), ((), ())),
                        preferred_element_type=jnp.float32)
    bm = jnp.max(x, axis=0, keepdims=True)
    bs = jnp.sum(jnp.exp(x - bm), axis=0, keepdims=True)
    m_old = m_ref[...]
    s_old = s_ref[...]
    m_new = jnp.maximum(m_old, bm)
    s_ref[...] = s_old * jnp.exp(m_old - m_new) + bs * jnp.exp(bm - m_new)
    m_ref[...] = m_new

    @pl.when(i == NBA - 1)
    def _fin():
        out_ref[pl.ds(l, 1), :] = m_ref[...] + jnp.log(s_ref[...])


def _out_kernel(emb_hbm, fcw_ref, lse_ref, out_ref, ebuf, esem):
    i = pl.program_id(0)
    for l in range(L):
        pltpu.make_async_copy(
            emb_hbm.at[pl.ds(l * B + i * BBLK, BBLK)], ebuf.at[l],
            esem.at[l]).start()
    fcw = fcw_ref[...]
    for l in range(L):
        pltpu.make_async_copy(
            emb_hbm.at[pl.ds(l * B + i * BBLK, BBLK)], ebuf.at[l],
            esem.at[l]).wait()
        xt = lax.dot_general(fcw, ebuf[l], (((1,), (1,)), ((), ())),
                             preferred_element_type=jnp.float32)
        lse_col = lse_ref[:, pl.ds(l, 1)]
        out_ref[pl.ds(l, 1), :, :] = (xt - lse_col)[None, :, :]


def kernel(input, emb_weight, fc_weight, fc_bias):
    idx_l = input.astype(jnp.int32).T.reshape(-1)
    idx3 = idx_l.reshape(NW, PER_W // IDXCOLS, IDXCOLS)
    table3 = emb_weight.reshape(-1, GRP, EMB)

    emb_l = _make_sc_gather()(table3, idx3)
    wt = fc_weight.T

    lse = pl.pallas_call(
        _lse_kernel,
        grid=(L, NBA),
        in_specs=[
            pl.BlockSpec((BBLKA, EMB), lambda l, i: (l * NBA + i, 0)),
            pl.BlockSpec((EMB, C), lambda l, i: (0, 0)),
        ],
        out_specs=pl.BlockSpec((L, C), lambda l, i: (0, 0)),
        out_shape=jax.ShapeDtypeStruct((L, C), jnp.float32),
        scratch_shapes=[
            pltpu.VMEM((1, C), jnp.float32),
            pltpu.VMEM((1, C), jnp.float32),
        ],
    )(emb_l, wt)

    out_t = pl.pallas_call(
        _out_kernel,
        grid=(NB,),
        in_specs=[
            pl.BlockSpec(memory_space=pl.ANY),
            pl.BlockSpec((C, EMB), lambda i: (0, 0)),
            pl.BlockSpec((C, L), lambda i: (0, 0)),
        ],
        out_specs=pl.BlockSpec((L, C, BBLK), lambda i: (0, 0, i)),
        out_shape=jax.ShapeDtypeStruct((L, C, B), jnp.float32),
        scratch_shapes=[
            pltpu.VMEM((L, BBLK, EMB), jnp.float32),
            pltpu.SemaphoreType.DMA((L,)),
        ],
    )(emb_l, fc_weight, lse.T)
    return jnp.transpose(out_t, (2, 0, 1))
